# baseline (speedup 1.0000x reference)
import functools
import math

import jax
import jax.numpy as jnp
from jax import lax
from jax.experimental import pallas as pl
from jax.experimental.pallas import tpu as pltpu
from jax.experimental.pallas import tpu_sc as plsc

N = 10000
D = 128
C = 40
L = 8
ALPHA = 0.5
THETA = 0.7

NC = 2
NS = 16
NW = NC * NS
CHUNK = 128
N_PAD = 10240
ROWS_PER_TILE = N_PAD // NS

BLK = 1024

_HIGH = lax.Precision.HIGHEST

@functools.lru_cache(maxsize=1)
def _sc_mesh():
    return plsc.VectorSubcoreMesh(core_axis_name="c", subcore_axis_name="s",
                                  num_cores=NC, num_subcores=NS)


def _gelu(v):
    return 0.5 * v * (1.0 + lax.erf(v * (1.0 / math.sqrt(2.0))))


def sc_degree(dst2d, nct):
    @functools.partial(
        pl.kernel,
        out_type=jax.ShapeDtypeStruct((NC, N_PAD, D), jnp.float32),
        mesh=_sc_mesh(),
        scratch_types=[
            pltpu.VMEM((nct, CHUNK), jnp.int32),
            pltpu.VMEM((CHUNK, D), jnp.float32),
            pltpu.VMEM((CHUNK, D), jnp.float32),
            pltpu.VMEM_SHARED((N_PAD, D), jnp.float32),
        ],
    )
    def k(dst_hbm, out_hbm, dst_v, zbuf, obuf, acc):
        c = lax.axis_index("c")
        s = lax.axis_index("s")
        wid = s * NC + c

        @pl.loop(0, CHUNK)
        def _(r):
            @pl.loop(0, D, step=16)
            def _(col):
                zbuf[r, pl.ds(col, 16)] = jnp.zeros((16,), jnp.float32)
                obuf[r, pl.ds(col, 16)] = jnp.ones((16,), jnp.float32)

        @pl.loop(0, ROWS_PER_TILE // CHUNK)
        def _(b):
            pltpu.sync_copy(zbuf,
                            acc.at[pl.ds(s * ROWS_PER_TILE + b * CHUNK, CHUNK)])

        pltpu.sync_copy(dst_hbm.at[pl.ds(wid * nct, nct)], dst_v)
        plsc.subcore_barrier()

        @pl.loop(0, nct)
        def _(j):
            pltpu.sync_copy(obuf, acc.at[dst_v.at[j]], add=True)

        plsc.subcore_barrier()

        @pl.loop(0, ROWS_PER_TILE // CHUNK)
        def _(b):
            off = s * ROWS_PER_TILE + b * CHUNK
            pltpu.sync_copy(acc.at[pl.ds(off, CHUNK)],
                            out_hbm.at[c, pl.ds(off, CHUNK)])

    return k(dst2d)


HALF = N_PAD // 2
TROWS = HALF // NS
ACC_PAD = HALF + CHUNK


def sc_aggregate(g2, srcg, dstg, nct_t):
    @functools.partial(
        pl.kernel,
        out_type=jax.ShapeDtypeStruct((NC, HALF, D), jnp.float32),
        mesh=_sc_mesh(),
        scratch_types=[
            pltpu.VMEM((nct_t, CHUNK), jnp.int32),
            pltpu.VMEM((nct_t, CHUNK), jnp.int32),
            pltpu.VMEM((CHUNK, D), jnp.float32),
            pltpu.VMEM((CHUNK, D), jnp.float32),
            pltpu.VMEM_SHARED((HALF, D), jnp.float32),
            pltpu.VMEM_SHARED((ACC_PAD, D), jnp.float32),
            pltpu.SemaphoreType.DMA,
            pltpu.SemaphoreType.DMA,
            pltpu.SemaphoreType.DMA,
            pltpu.SemaphoreType.DMA,
        ],
    )
    def k(g_hbm, src_hbm, dst_hbm, out_hbm, src_v, dst_v, b0, b1, tbl, acc,
          semg0, semg1, sems0, sems1):
        c = lax.axis_index("c")
        s = lax.axis_index("s")

        tl = pltpu.async_copy(g_hbm.at[c, pl.ds(s * TROWS, TROWS)],
                              tbl.at[pl.ds(s * TROWS, TROWS)], semg1)
        tile = c * NS + s
        il0 = pltpu.async_copy(src_hbm.at[tile], src_v, sems0)
        il1 = pltpu.async_copy(dst_hbm.at[tile], dst_v, sems1)

        @pl.loop(0, CHUNK)
        def _(r):
            @pl.loop(0, D, step=16)
            def _(col):
                b0[r, pl.ds(col, 16)] = jnp.zeros((16,), jnp.float32)

        pltpu.sync_copy(b0, acc.at[pl.ds(s * TROWS, CHUNK)])
        pltpu.sync_copy(b0, acc.at[pl.ds(s * TROWS + CHUNK, CHUNK)])
        pltpu.sync_copy(b0.at[pl.ds(0, 64)],
                        acc.at[pl.ds(s * TROWS + 2 * CHUNK, 64)])
        tl.wait()
        il0.wait()
        il1.wait()
        plsc.subcore_barrier()

        @pl.loop(0, nct_t, step=2)
        def _(j):
            c0 = pltpu.async_copy(tbl.at[src_v.at[j]], b0, semg0)
            c1 = pltpu.async_copy(tbl.at[src_v.at[j + 1]], b1, semg1)
            c0.wait()
            s0 = pltpu.async_copy(b0, acc.at[dst_v.at[j]], sems0, add=True)
            c1.wait()
            s1 = pltpu.async_copy(b1, acc.at[dst_v.at[j + 1]], sems1, add=True)
            s0.wait()
            s1.wait()

        plsc.subcore_barrier()

        pltpu.sync_copy(acc.at[pl.ds(s * TROWS, TROWS)],
                        out_hbm.at[c, pl.ds(s * TROWS, TROWS)])

    return k(g2, srcg, dstg)


def tc_lin1(x_pad, w_t, b):
    def body(x_ref, w_ref, b_ref, o_ref):
        o_ref[...] = jnp.dot(x_ref[...], w_ref[...],
                             preferred_element_type=jnp.float32,
                             precision=_HIGH) + b_ref[...]

    return pl.pallas_call(
        body,
        grid=(N_PAD // BLK,),
        in_specs=[
            pl.BlockSpec((BLK, D), lambda i: (i, 0)),
            pl.BlockSpec((D, D), lambda i: (0, 0)),
            pl.BlockSpec((1, D), lambda i: (0, 0)),
        ],
        out_specs=pl.BlockSpec((BLK, D), lambda i: (i, 0)),
        out_shape=jax.ShapeDtypeStruct((N_PAD, D), jnp.float32),
    )(x_pad, w_t, b)


NBH = HALF // BLK


def tc_prep(deg2, h0):
    def body(deg_ref, h_ref, dinv_ref, g_ref):
        i = pl.program_id(0)
        deg = deg_ref[0, :, 0:1] + deg_ref[1, :, 0:1] + 1.0
        rows = i * BLK + lax.broadcasted_iota(jnp.int32, (BLK, 1), 0)
        dinv = jnp.where(rows < N, lax.rsqrt(deg), 0.0)
        dinvb = jnp.broadcast_to(dinv, (BLK, D))
        dinv_ref[...] = dinvb
        g_ref[0] = h_ref[...] * dinvb

    return pl.pallas_call(
        body,
        grid=(N_PAD // BLK,),
        in_specs=[
            pl.BlockSpec((NC, BLK, D), lambda i: (0, i, 0)),
            pl.BlockSpec((BLK, D), lambda i: (i, 0)),
        ],
        out_specs=[
            pl.BlockSpec((BLK, D), lambda i: (i, 0)),
            pl.BlockSpec((1, BLK, D), lambda i: (i // NBH, i % NBH, 0)),
        ],
        out_shape=[
            jax.ShapeDtypeStruct((N_PAD, D), jnp.float32),
            jax.ShapeDtypeStruct((NC, HALF, D), jnp.float32),
        ],
    )(deg2, h0)


def _layer_math(sblk, g, x0, dinv, w, beta):
    agg = (sblk + g) * dinv
    out = (1.0 - ALPHA) * agg + ALPHA * x0
    t = jnp.dot(out, w, preferred_element_type=jnp.float32, precision=_HIGH)
    out = (1.0 - beta) * out + beta * t
    return _gelu(out)


_HALF_IN_SPECS = [
    pl.BlockSpec((NC, BLK, D), lambda i: (0, jnp.minimum(i, NBH - 1), 0)),
    pl.BlockSpec((NC, BLK, D), lambda i: (0, jnp.maximum(i - NBH, 0), 0)),
    pl.BlockSpec((1, BLK, D), lambda i: (i // NBH, i % NBH, 0)),
    pl.BlockSpec((BLK, D), lambda i: (i, 0)),
    pl.BlockSpec((BLK, D), lambda i: (i, 0)),
    pl.BlockSpec((D, D), lambda i: (0, 0)),
]


def _merged_s(lo_ref, hi_ref):
    i = pl.program_id(0)
    lo = lo_ref[0] + lo_ref[1]
    hi = hi_ref[0] + hi_ref[1]
    return jnp.where(i < NBH, lo, hi)


def tc_layer(s_lo, s_hi, g, x0, dinvb, w, beta):
    def body(lo_ref, hi_ref, g_ref, x0_ref, dinv_ref, w_ref, g_next_ref):
        h = _layer_math(_merged_s(lo_ref, hi_ref), g_ref[0], x0_ref[...],
                        dinv_ref[...], w_ref[...], beta)
        g_next_ref[0] = h * dinv_ref[...]

    return pl.pallas_call(
        body,
        grid=(N_PAD // BLK,),
        in_specs=_HALF_IN_SPECS,
        out_specs=pl.BlockSpec((1, BLK, D), lambda i: (i // NBH, i % NBH, 0)),
        out_shape=jax.ShapeDtypeStruct((NC, HALF, D), jnp.float32),
    )(s_lo, s_hi, g, x0, dinvb, w)


def tc_layer_final(s_lo, s_hi, g, x0, dinvb, w, beta,
                   ln_g, ln_b, fc1_wt, fc1_b, fc2_wt, fc2_b):
    def body(lo_ref, hi_ref, g_ref, x0_ref, dinv_ref, w_ref,
             lng_ref, lnb_ref, w1_ref, b1_ref, w2_ref, b2_ref, o_ref):
        h = _layer_math(_merged_s(lo_ref, hi_ref), g_ref[0], x0_ref[...],
                        dinv_ref[...], w_ref[...], beta)
        mu = jnp.mean(h, axis=-1, keepdims=True)
        xc = h - mu
        var = jnp.mean(xc * xc, axis=-1, keepdims=True)
        hn = xc * lax.rsqrt(var + 1e-5) * lng_ref[...] + lnb_ref[...]
        h2 = _gelu(hn)
        h3 = _gelu(jnp.dot(h2, w1_ref[...], preferred_element_type=jnp.float32,
                           precision=_HIGH) + b1_ref[...])
        o_ref[...] = jnp.dot(h3, w2_ref[...], preferred_element_type=jnp.float32,
                             precision=_HIGH) + b2_ref[...]

    return pl.pallas_call(
        body,
        grid=(N_PAD // BLK,),
        in_specs=_HALF_IN_SPECS + [
            pl.BlockSpec((1, D), lambda i: (0, 0)),
            pl.BlockSpec((1, D), lambda i: (0, 0)),
            pl.BlockSpec((D, D), lambda i: (0, 0)),
            pl.BlockSpec((1, D), lambda i: (0, 0)),
            pl.BlockSpec((D, C), lambda i: (0, 0)),
            pl.BlockSpec((1, C), lambda i: (0, 0)),
        ],
        out_specs=pl.BlockSpec((BLK, C), lambda i: (i, 0)),
        out_shape=jax.ShapeDtypeStruct((N_PAD, C), jnp.float32),
    )(s_lo, s_hi, g, x0, dinvb, w, ln_g, ln_b, fc1_wt, fc1_b, fc2_wt, fc2_b)


def kernel(x, edge_index, edge_attr, lin1_W, lin1_b, conv_W, ln_g, ln_b,
           fc1_W, fc1_b, fc2_W, fc2_b):
    del edge_attr
    x = x.astype(jnp.float32)
    src = edge_index[0].astype(jnp.int32)
    dst = edge_index[1].astype(jnp.int32)
    e = src.shape[0]

    nct = -(-e // (NW * CHUNK))
    if nct % 2:
        nct += 1
    e_pad = NW * CHUNK * nct
    pad = jnp.full((e_pad - e,), N, jnp.int32)
    dst2d = jnp.concatenate([dst, pad]).reshape(NW * nct, CHUNK)

    q = (src >= HALF).astype(jnp.int32) * 2 + (dst >= HALF).astype(jnp.int32)
    order = jnp.argsort(q)
    src_s = src[order]
    dst_s = dst[order]
    counts = jnp.bincount(q, length=4)
    starts = jnp.cumsum(counts) - counts
    sigma = math.sqrt(e * 0.25 * 0.75)
    nct_t = -(-int(e / 4 + 24.0 * sigma) // (NS * CHUNK))
    if nct_t % 2:
        nct_t += 1
    cap = NS * CHUNK * nct_t
    i4 = jnp.arange(4 * cap, dtype=jnp.int32)
    gi = i4 // cap
    p = i4 % cap
    valid = p < counts[gi]
    jc = jnp.minimum(starts[gi] + p, e - 1)
    srel = jnp.where(valid, src_s[jc] - (gi >= 2) * HALF, 0).astype(jnp.int32)
    drel = jnp.where(valid, dst_s[jc] - (gi % 2) * HALF,
                     HALF).astype(jnp.int32)
    ps = srel.reshape(4, NS, nct_t, CHUNK)
    pd = drel.reshape(4, NS, nct_t, CHUNK)
    src_lo = jnp.concatenate([ps[0], ps[2]])
    dst_lo = jnp.concatenate([pd[0], pd[2]])
    src_hi = jnp.concatenate([ps[1], ps[3]])
    dst_hi = jnp.concatenate([pd[1], pd[3]])

    x_pad = jnp.pad(x, ((0, N_PAD - N), (0, 0)))

    deg2 = sc_degree(dst2d, nct)
    h0 = tc_lin1(x_pad, lin1_W.T, lin1_b.reshape(1, D))
    dinvb, g2 = tc_prep(deg2, h0)
    x0 = h0

    logits = None
    for i in range(L):
        beta = float(math.log(THETA / (i + 1) + 1.0))
        s_lo = sc_aggregate(g2, src_lo, dst_lo, nct_t)
        s_hi = sc_aggregate(g2, src_hi, dst_hi, nct_t)
        if i < L - 1:
            g2 = tc_layer(s_lo, s_hi, g2, x0, dinvb, conv_W[i], beta)
        else:
            logits = tc_layer_final(
                s_lo, s_hi, g2, x0, dinvb, conv_W[i], beta,
                ln_g.reshape(1, D), ln_b.reshape(1, D),
                fc1_W.T, fc1_b.reshape(1, D),
                fc2_W.T, fc2_b.reshape(1, C))
    return logits[:N]


# speedup vs baseline: 1.0515x; 1.0515x over previous
import functools
import math

import jax
import jax.numpy as jnp
from jax import lax
from jax.experimental import pallas as pl
from jax.experimental.pallas import tpu as pltpu
from jax.experimental.pallas import tpu_sc as plsc

N = 10000
D = 128
C = 40
L = 8
ALPHA = 0.5
THETA = 0.7

NC = 2
NS = 16
NW = NC * NS
CHUNK = 128
N_PAD = 10240
ROWS_PER_TILE = N_PAD // NS

BLK = 1024

_HIGH = lax.Precision.HIGHEST

@functools.lru_cache(maxsize=1)
def _sc_mesh():
    return plsc.VectorSubcoreMesh(core_axis_name="c", subcore_axis_name="s",
                                  num_cores=NC, num_subcores=NS)


def _gelu(v):
    return 0.5 * v * (1.0 + lax.erf(v * (1.0 / math.sqrt(2.0))))


def sc_degree(dst2d, nct):
    @functools.partial(
        pl.kernel,
        out_type=jax.ShapeDtypeStruct((NC, N_PAD, D), jnp.float32),
        mesh=_sc_mesh(),
        scratch_types=[
            pltpu.VMEM((nct, CHUNK), jnp.int32),
            pltpu.VMEM((CHUNK, D), jnp.float32),
            pltpu.VMEM((CHUNK, D), jnp.float32),
            pltpu.VMEM_SHARED((N_PAD, D), jnp.float32),
        ],
    )
    def k(dst_hbm, out_hbm, dst_v, zbuf, obuf, acc):
        c = lax.axis_index("c")
        s = lax.axis_index("s")
        wid = s * NC + c

        @pl.loop(0, CHUNK)
        def _(r):
            @pl.loop(0, D, step=16)
            def _(col):
                zbuf[r, pl.ds(col, 16)] = jnp.zeros((16,), jnp.float32)
                obuf[r, pl.ds(col, 16)] = jnp.ones((16,), jnp.float32)

        @pl.loop(0, ROWS_PER_TILE // CHUNK)
        def _(b):
            pltpu.sync_copy(zbuf,
                            acc.at[pl.ds(s * ROWS_PER_TILE + b * CHUNK, CHUNK)])

        pltpu.sync_copy(dst_hbm.at[pl.ds(wid * nct, nct)], dst_v)
        plsc.subcore_barrier()

        @pl.loop(0, nct)
        def _(j):
            pltpu.sync_copy(obuf, acc.at[dst_v.at[j]], add=True)

        plsc.subcore_barrier()

        @pl.loop(0, ROWS_PER_TILE // CHUNK)
        def _(b):
            off = s * ROWS_PER_TILE + b * CHUNK
            pltpu.sync_copy(acc.at[pl.ds(off, CHUNK)],
                            out_hbm.at[c, pl.ds(off, CHUNK)])

    return k(dst2d)


HALF = N_PAD // 2
TROWS = HALF // NS
ACC_PAD = HALF + CHUNK


def sc_aggregate(g2, srcg, dstg, nct_t):
    @functools.partial(
        pl.kernel,
        out_type=jax.ShapeDtypeStruct((NC, N_PAD, D), jnp.float32),
        mesh=_sc_mesh(),
        scratch_types=[
            pltpu.VMEM((nct_t, CHUNK), jnp.int32),
            pltpu.VMEM((nct_t, CHUNK), jnp.int32),
            pltpu.VMEM((CHUNK, D), jnp.float32),
            pltpu.VMEM((CHUNK, D), jnp.float32),
            pltpu.VMEM_SHARED((HALF, D), jnp.float32),
            pltpu.VMEM_SHARED((ACC_PAD, D), jnp.float32),
            pltpu.SemaphoreType.DMA,
            pltpu.SemaphoreType.DMA,
            pltpu.SemaphoreType.DMA,
            pltpu.SemaphoreType.DMA,
        ],
    )
    def k(g_hbm, src_hbm, dst_hbm, out_hbm, src_v, dst_v, b0, b1, tbl, acc,
          semg0, semg1, sems0, sems1):
        c = lax.axis_index("c")
        s = lax.axis_index("s")
        tile = c * NS + s

        tl = pltpu.async_copy(g_hbm.at[c, pl.ds(s * TROWS, TROWS)],
                              tbl.at[pl.ds(s * TROWS, TROWS)], semg1)
        il0 = pltpu.async_copy(src_hbm.at[0, tile], src_v, sems0)
        il1 = pltpu.async_copy(dst_hbm.at[0, tile], dst_v, sems1)

        @pl.loop(0, CHUNK)
        def _(r):
            @pl.loop(0, D, step=16)
            def _(col):
                b0[r, pl.ds(col, 16)] = jnp.zeros((16,), jnp.float32)

        def zero_my_rows():
            pltpu.sync_copy(b0, acc.at[pl.ds(s * TROWS, CHUNK)])
            pltpu.sync_copy(b0, acc.at[pl.ds(s * TROWS + CHUNK, CHUNK)])
            pltpu.sync_copy(b0.at[pl.ds(0, 64)],
                            acc.at[pl.ds(s * TROWS + 2 * CHUNK, 64)])

        def edge_loop():
            @pl.loop(0, nct_t, step=2)
            def _(j):
                c0 = pltpu.async_copy(tbl.at[src_v.at[j]], b0, semg0)
                c1 = pltpu.async_copy(tbl.at[src_v.at[j + 1]], b1, semg1)
                c0.wait()
                s0 = pltpu.async_copy(b0, acc.at[dst_v.at[j]], sems0,
                                      add=True)
                c1.wait()
                s1 = pltpu.async_copy(b1, acc.at[dst_v.at[j + 1]], sems1,
                                      add=True)
                s0.wait()
                s1.wait()

        zero_my_rows()
        tl.wait()
        il0.wait()
        il1.wait()
        plsc.subcore_barrier()

        edge_loop()

        plsc.subcore_barrier()
        pltpu.sync_copy(acc.at[pl.ds(s * TROWS, TROWS)],
                        out_hbm.at[c, pl.ds(s * TROWS, TROWS)])
        il0b = pltpu.async_copy(src_hbm.at[1, tile], src_v, sems0)
        il1b = pltpu.async_copy(dst_hbm.at[1, tile], dst_v, sems1)

        @pl.loop(0, CHUNK)
        def _(r):
            @pl.loop(0, D, step=16)
            def _(col):
                b0[r, pl.ds(col, 16)] = jnp.zeros((16,), jnp.float32)

        zero_my_rows()
        il0b.wait()
        il1b.wait()
        plsc.subcore_barrier()

        edge_loop()

        plsc.subcore_barrier()
        pltpu.sync_copy(acc.at[pl.ds(s * TROWS, TROWS)],
                        out_hbm.at[c, pl.ds(HALF + s * TROWS, TROWS)])

    return k(g2, srcg, dstg)


def tc_lin1(x_pad, w_t, b):
    def body(x_ref, w_ref, b_ref, o_ref):
        o_ref[...] = jnp.dot(x_ref[...], w_ref[...],
                             preferred_element_type=jnp.float32,
                             precision=_HIGH) + b_ref[...]

    return pl.pallas_call(
        body,
        grid=(N_PAD // BLK,),
        in_specs=[
            pl.BlockSpec((BLK, D), lambda i: (i, 0)),
            pl.BlockSpec((D, D), lambda i: (0, 0)),
            pl.BlockSpec((1, D), lambda i: (0, 0)),
        ],
        out_specs=pl.BlockSpec((BLK, D), lambda i: (i, 0)),
        out_shape=jax.ShapeDtypeStruct((N_PAD, D), jnp.float32),
    )(x_pad, w_t, b)


NBH = HALF // BLK


def tc_prep(deg2, h0):
    def body(deg_ref, h_ref, dinv_ref, g_ref):
        i = pl.program_id(0)
        deg = deg_ref[0, :, 0:1] + deg_ref[1, :, 0:1] + 1.0
        rows = i * BLK + lax.broadcasted_iota(jnp.int32, (BLK, 1), 0)
        dinv = jnp.where(rows < N, lax.rsqrt(deg), 0.0)
        dinvb = jnp.broadcast_to(dinv, (BLK, D))
        dinv_ref[...] = dinvb
        g_ref[0] = h_ref[...] * dinvb

    return pl.pallas_call(
        body,
        grid=(N_PAD // BLK,),
        in_specs=[
            pl.BlockSpec((NC, BLK, D), lambda i: (0, i, 0)),
            pl.BlockSpec((BLK, D), lambda i: (i, 0)),
        ],
        out_specs=[
            pl.BlockSpec((BLK, D), lambda i: (i, 0)),
            pl.BlockSpec((1, BLK, D), lambda i: (i // NBH, i % NBH, 0)),
        ],
        out_shape=[
            jax.ShapeDtypeStruct((N_PAD, D), jnp.float32),
            jax.ShapeDtypeStruct((NC, HALF, D), jnp.float32),
        ],
    )(deg2, h0)


def _layer_math(sblk, g, x0, dinv, w, beta):
    agg = (sblk + g) * dinv
    out = (1.0 - ALPHA) * agg + ALPHA * x0
    t = jnp.dot(out, w, preferred_element_type=jnp.float32, precision=_HIGH)
    out = (1.0 - beta) * out + beta * t
    return _gelu(out)


_LAYER_IN_SPECS = [
    pl.BlockSpec((NC, BLK, D), lambda i: (0, i, 0)),
    pl.BlockSpec((1, BLK, D), lambda i: (i // NBH, i % NBH, 0)),
    pl.BlockSpec((BLK, D), lambda i: (i, 0)),
    pl.BlockSpec((BLK, D), lambda i: (i, 0)),
    pl.BlockSpec((D, D), lambda i: (0, 0)),
]


def tc_layer(s2, g, x0, dinvb, w, beta):
    def body(s_ref, g_ref, x0_ref, dinv_ref, w_ref, g_next_ref):
        h = _layer_math(s_ref[0] + s_ref[1], g_ref[0], x0_ref[...],
                        dinv_ref[...], w_ref[...], beta)
        g_next_ref[0] = h * dinv_ref[...]

    return pl.pallas_call(
        body,
        grid=(N_PAD // BLK,),
        in_specs=_LAYER_IN_SPECS,
        out_specs=pl.BlockSpec((1, BLK, D), lambda i: (i // NBH, i % NBH, 0)),
        out_shape=jax.ShapeDtypeStruct((NC, HALF, D), jnp.float32),
    )(s2, g, x0, dinvb, w)


def tc_layer_final(s2, g, x0, dinvb, w, beta,
                   ln_g, ln_b, fc1_wt, fc1_b, fc2_wt, fc2_b):
    def body(s_ref, g_ref, x0_ref, dinv_ref, w_ref,
             lng_ref, lnb_ref, w1_ref, b1_ref, w2_ref, b2_ref, o_ref):
        h = _layer_math(s_ref[0] + s_ref[1], g_ref[0], x0_ref[...],
                        dinv_ref[...], w_ref[...], beta)
        mu = jnp.mean(h, axis=-1, keepdims=True)
        xc = h - mu
        var = jnp.mean(xc * xc, axis=-1, keepdims=True)
        hn = xc * lax.rsqrt(var + 1e-5) * lng_ref[...] + lnb_ref[...]
        h2 = _gelu(hn)
        h3 = _gelu(jnp.dot(h2, w1_ref[...], preferred_element_type=jnp.float32,
                           precision=_HIGH) + b1_ref[...])
        o_ref[...] = jnp.dot(h3, w2_ref[...], preferred_element_type=jnp.float32,
                             precision=_HIGH) + b2_ref[...]

    return pl.pallas_call(
        body,
        grid=(N_PAD // BLK,),
        in_specs=_LAYER_IN_SPECS + [
            pl.BlockSpec((1, D), lambda i: (0, 0)),
            pl.BlockSpec((1, D), lambda i: (0, 0)),
            pl.BlockSpec((D, D), lambda i: (0, 0)),
            pl.BlockSpec((1, D), lambda i: (0, 0)),
            pl.BlockSpec((D, C), lambda i: (0, 0)),
            pl.BlockSpec((1, C), lambda i: (0, 0)),
        ],
        out_specs=pl.BlockSpec((BLK, C), lambda i: (i, 0)),
        out_shape=jax.ShapeDtypeStruct((N_PAD, C), jnp.float32),
    )(s2, g, x0, dinvb, w, ln_g, ln_b, fc1_wt, fc1_b, fc2_wt, fc2_b)


def kernel(x, edge_index, edge_attr, lin1_W, lin1_b, conv_W, ln_g, ln_b,
           fc1_W, fc1_b, fc2_W, fc2_b):
    del edge_attr
    x = x.astype(jnp.float32)
    src = edge_index[0].astype(jnp.int32)
    dst = edge_index[1].astype(jnp.int32)
    e = src.shape[0]

    nct = -(-e // (NW * CHUNK))
    if nct % 2:
        nct += 1
    e_pad = NW * CHUNK * nct
    pad = jnp.full((e_pad - e,), N, jnp.int32)
    dst2d = jnp.concatenate([dst, pad]).reshape(NW * nct, CHUNK)

    q = (src >= HALF).astype(jnp.int32) * 2 + (dst >= HALF).astype(jnp.int32)
    order = jnp.argsort(q)
    src_s = src[order]
    dst_s = dst[order]
    starts = jnp.searchsorted(q[order], jnp.arange(4, dtype=q.dtype))
    counts = jnp.diff(jnp.concatenate(
        [starts, jnp.array([e], starts.dtype)]))
    sigma = math.sqrt(e * 0.25 * 0.75)
    nct_t = -(-int(e / 4 + 24.0 * sigma) // (NS * CHUNK))
    if nct_t % 2:
        nct_t += 1
    cap = NS * CHUNK * nct_t
    i4 = jnp.arange(4 * cap, dtype=jnp.int32)
    gi = i4 // cap
    p = i4 % cap
    valid = p < counts[gi]
    jc = jnp.minimum(starts[gi] + p, e - 1)
    srel = jnp.where(valid, src_s[jc] - (gi >= 2) * HALF, 0).astype(jnp.int32)
    drel = jnp.where(valid, dst_s[jc] - (gi % 2) * HALF,
                     HALF).astype(jnp.int32)
    ps = srel.reshape(2, 2, NS, nct_t, CHUNK).transpose(1, 0, 2, 3, 4)
    pd = drel.reshape(2, 2, NS, nct_t, CHUNK).transpose(1, 0, 2, 3, 4)
    src_all = ps.reshape(2, NC * NS, nct_t, CHUNK)
    dst_all = pd.reshape(2, NC * NS, nct_t, CHUNK)

    x_pad = jnp.pad(x, ((0, N_PAD - N), (0, 0)))

    deg2 = sc_degree(dst2d, nct)
    h0 = tc_lin1(x_pad, lin1_W.T, lin1_b.reshape(1, D))
    dinvb, g2 = tc_prep(deg2, h0)
    x0 = h0

    logits = None
    for i in range(L):
        beta = float(math.log(THETA / (i + 1) + 1.0))
        s2 = sc_aggregate(g2, src_all, dst_all, nct_t)
        if i < L - 1:
            g2 = tc_layer(s2, g2, x0, dinvb, conv_W[i], beta)
        else:
            logits = tc_layer_final(
                s2, g2, x0, dinvb, conv_W[i], beta,
                ln_g.reshape(1, D), ln_b.reshape(1, D),
                fc1_W.T, fc1_b.reshape(1, D),
                fc2_W.T, fc2_b.reshape(1, C))
    return logits[:N]
